# Initial kernel scaffold; baseline (speedup 1.0000x reference)
import jax
import jax.numpy as jnp
from jax.experimental import pallas as pl


def kernel(temporal_block, revert_idx, mask_token):
    raise NotImplementedError("write your pallas kernel here")


# speedup vs baseline: 2.8641x; 2.8641x over previous
import functools

import jax
import jax.numpy as jnp
from jax import lax
from jax.experimental import pallas as pl
from jax.experimental.pallas import tpu as pltpu
from jax.experimental.pallas import tpu_sc as plsc

B, S, D = 512, 50, 64
NMOD = 14
NSLOT = 26
NOUT = 27
PAIRS = B * S

NC, NS = 2, 16
NW = NC * NS
PPW = PAIRS // NW
CP = 16
NCH = PPW // CP
ROWS = CP * NOUT
NG = ROWS // 16
INW = CP * NMOD * D
MASKROW = CP * NMOD


def _body(tb, ri, mt, out, idx_v, in_v, out_v, mt_v, sem):
    wid = lax.axis_index("s") * NC + lax.axis_index("c")
    iota = lax.iota(jnp.int32, 16)

    pltpu.sync_copy(mt, mt_v)
    for k in range(4):
        in_v[pl.ds(MASKROW * D + k * 16, 16)] = mt_v[pl.ds(k * 16, 16)]

    def chunk_body(c, carry):
        pair0 = pl.multiple_of(wid * PPW + c * CP, CP)
        pltpu.sync_copy(ri.at[pl.ds(pair0 * NSLOT, CP * NSLOT)], idx_v)
        pltpu.sync_copy(
            tb.at[pl.ds(pair0 * NMOD * D, INW)], in_v.at[pl.ds(0, INW)]
        )

        def group_body(g, gc):
            r = iota + g * 16
            p_rel = lax.div(r, NOUT)
            n = r - p_rel * NOUT
            off = jnp.maximum(p_rel * NSLOT + n - 1, 0)
            idx = plsc.load_gather(idx_v, [off])
            keep = idx < (NMOD - 1)
            base = p_rel * NMOD
            src_rel = jnp.where(
                n == 0, base, jnp.where(keep, base + 1 + idx, MASKROW)
            )
            srcf = src_rel * D
            dst0 = g * 16 * D
            for i in range(16):
                s0 = srcf[i]
                d0 = dst0 + i * D
                for k in range(4):
                    out_v[pl.ds(d0 + k * 16, 16)] = in_v[pl.ds(s0 + k * 16, 16)]
            return gc
        lax.fori_loop(0, NG, group_body, 0)

        pltpu.sync_copy(
            out_v, out.at[pl.ds(pair0 * NOUT * D, ROWS * D)]
        )
        return carry

    lax.fori_loop(0, NCH, chunk_body, 0)


_revert = functools.partial(
    pl.kernel,
    _body,
    out_type=jax.ShapeDtypeStruct((PAIRS * NOUT * D,), jnp.float32),
    mesh=plsc.VectorSubcoreMesh(core_axis_name="c", subcore_axis_name="s"),
    compiler_params=pltpu.CompilerParams(needs_layout_passes=False),
    scratch_types=[
        pltpu.VMEM((CP * NSLOT,), jnp.int32),
        pltpu.VMEM((INW + D,), jnp.float32),
        pltpu.VMEM((ROWS * D,), jnp.float32),
        pltpu.VMEM((D,), jnp.float32),
        pltpu.SemaphoreType.DMA,
    ],
)


def kernel(temporal_block, revert_idx, mask_token):
    tb = temporal_block.reshape(PAIRS * NMOD * D)
    ri = revert_idx.reshape(PAIRS * NSLOT)
    out = _revert()(tb, ri, mask_token)
    return out.reshape(B, S, NOUT, D)
